# Initial kernel scaffold; baseline (speedup 1.0000x reference)
import jax
import jax.numpy as jnp
from jax.experimental import pallas as pl


def kernel(drug1_neighbors, drug2_neighbors, cell_neighbors, rels, protein_emb, rel_emb, W_agg, b_agg, W_rel, b_rel, ln_gamma, ln_beta):
    raise NotImplementedError("write your pallas kernel here")


# speedup vs baseline: 4.2592x; 4.2592x over previous
import functools

import jax
import jax.numpy as jnp
from jax import lax
from jax.experimental import pallas as pl
from jax.experimental.pallas import tpu as pltpu
from jax.experimental.pallas import tpu_sc as plsc

B = 1024
NHOP = 2
NMEM = 64
D = 128
NB = NHOP * NMEM
L = 16
ND = D // L
NC = 2
NS = 16
NW = NC * NS
ROWS_PER_W = B // NW
NGROUP = NMEM // L


def _rsqrt_vec(v):
    i = lax.bitcast_convert_type(v, jnp.int32)
    i = jnp.int32(0x5F3759DF) - lax.shift_right_logical(i, 1)
    y = lax.bitcast_convert_type(i, jnp.float32)
    for _ in range(3):
        y = y * (1.5 - 0.5 * v * y * y)
    return y


def _row_vecs(ref, n):
    return [ref[n, pl.ds(j * L, L)] for j in range(ND)]


def _splat(s):
    return jnp.full((L,), s, dtype=jnp.float32)


def _all_reduce(v, red_v, op):
    iot = lax.iota(jnp.int32, L)
    for s in (1, 2, 4, 8):
        red_v[...] = v
        v = op(v, plsc.load_gather(red_v, [jnp.bitwise_xor(iot, s)]))
    return v


def _sc_body(cidx_hbm, d1idx_hbm, d2idx_hbm, rels_hbm, prot_hbm, rele_hbm,
             gam_hbm, bet_hbm,
             feat1_hbm, feat2_hbm, r_hbm,
             cidx_v, d1idx_v, d2idx_v, ec_v, e1_v, e2_v,
             gb_v, fbuf_v, tbuf_v, abuf_v, rels_v, rrow_v, red_v,
             sem_c, sem_1, sem_2, sem_r):
    wid = lax.axis_index("s") * NC + lax.axis_index("c")
    base_row = wid * ROWS_PER_W

    pltpu.sync_copy(rels_hbm.at[pl.ds(base_row, ROWS_PER_W)], rels_v)
    pltpu.async_copy(rele_hbm.at[rels_v], rrow_v, sem_r).wait()
    pltpu.sync_copy(rrow_v, r_hbm.at[pl.ds(base_row, ROWS_PER_W)])

    pltpu.sync_copy(gam_hbm, gb_v.at[0])
    pltpu.sync_copy(bet_hbm, gb_v.at[1])
    g_vecs = [gb_v[0, pl.ds(j * L, L)] for j in range(ND)]
    bt_vecs = [gb_v[1, pl.ds(j * L, L)] for j in range(ND)]

    def process_hop(e_ref, h, ctx):
        base = h * NMEM
        iot = lax.iota(jnp.int32, L)
        lg = []
        for g in range(NGROUP):
            def k_body(k, c, g=g):
                row = _row_vecs(e_ref, base + g * L + k)
                t = row[0] * ctx[0]
                for j in range(1, ND):
                    t = t + row[j] * ctx[j]
                tbuf_v[k, :] = t
                return c
            lax.fori_loop(0, L, k_body, 0)

            def c_body(cc, acc):
                col = plsc.load_gather(tbuf_v, [iot, jnp.full((L,), cc, jnp.int32)])
                return acc + col
            lg.append(lax.fori_loop(0, L, c_body, jnp.zeros((L,), jnp.float32)))
        mx = jnp.maximum(jnp.maximum(lg[0], lg[1]), jnp.maximum(lg[2], lg[3]))
        mxs = _all_reduce(mx, red_v, jnp.maximum)
        ex = [jnp.exp(v - mxs) for v in lg]
        ssum = _all_reduce(ex[0] + ex[1] + ex[2] + ex[3], red_v, jnp.add)
        for g in range(NGROUP):
            abuf_v[pl.ds(g * L, L)] = ex[g] / ssum
        def ws_body(n, carry):
            row = _row_vecs(e_ref, base + n)
            av = _splat(abuf_v[pl.ds(n, L)][0])
            o = [carry[j] + av * row[j] for j in range(ND)]
            m = [carry[ND + j] + row[j] for j in range(ND)]
            return tuple(o) + tuple(m)
        init = tuple(jnp.zeros((L,), jnp.float32) for _ in range(2 * ND))
        res = lax.fori_loop(0, NMEM, ws_body, init, unroll=2)
        o = res[:ND]
        m = [v * (1.0 / NMEM) for v in res[ND:]]
        return o, m

    def row_body(i, c):
        b = base_row + i
        pltpu.sync_copy(cidx_hbm.at[b], cidx_v)
        pltpu.sync_copy(d1idx_hbm.at[b], d1idx_v)
        pltpu.sync_copy(d2idx_hbm.at[b], d2idx_v)
        cp_c = pltpu.async_copy(prot_hbm.at[cidx_v], ec_v, sem_c)
        cp_1 = pltpu.async_copy(prot_hbm.at[d1idx_v], e1_v, sem_1)
        cp_2 = pltpu.async_copy(prot_hbm.at[d2idx_v], e2_v, sem_2)
        cp_c.wait()

        def cell_body(n, acc):
            row = _row_vecs(ec_v, n)
            return tuple(a + r for a, r in zip(acc, row))
        acc = lax.fori_loop(0, NB, cell_body,
                            tuple(jnp.zeros((L,), jnp.float32) for _ in range(ND)),
                            unroll=2)
        x = [a * (1.0 / NB) for a in acc]
        t = x[0]
        t2 = x[0] * x[0]
        for j in range(1, ND):
            t = t + x[j]
            t2 = t2 + x[j] * x[j]
        muv = _all_reduce(t, red_v, jnp.add) * (1.0 / D)
        varv = _all_reduce(t2, red_v, jnp.add) * (1.0 / D) - muv * muv
        rstd = _rsqrt_vec(varv + 1e-5)
        ctx = [g_vecs[j] * (x[j] - muv) * rstd + bt_vecs[j] for j in range(ND)]

        for e_ref, cp, feat_hbm in ((e1_v, cp_1, feat1_hbm), (e2_v, cp_2, feat2_hbm)):
            cp.wait()
            for h in range(NHOP):
                o, m = process_hop(e_ref, h, ctx)
                off = h * 2 * D
                for j in range(ND):
                    fbuf_v[pl.ds(off + j * L, L)] = o[j]
                    fbuf_v[pl.ds(off + D + j * L, L)] = m[j]
            pltpu.sync_copy(fbuf_v, feat_hbm.at[b])
        return c

    lax.fori_loop(0, ROWS_PER_W, row_body, 0)


@functools.partial(jax.jit, static_argnames=())
def _sc_call(cidx, d1idx, d2idx, rels, prot, rele, gam, bet):
    mesh = plsc.VectorSubcoreMesh(core_axis_name="c", subcore_axis_name="s")
    f = pl.kernel(
        _sc_body,
        compiler_params=pltpu.CompilerParams(needs_layout_passes=False),
        out_type=[
            jax.ShapeDtypeStruct((B, 2 * NHOP * D), jnp.float32),
            jax.ShapeDtypeStruct((B, 2 * NHOP * D), jnp.float32),
            jax.ShapeDtypeStruct((B, D), jnp.float32),
        ],
        mesh=mesh,
        scratch_types=[
            pltpu.VMEM((NB,), jnp.int32),
            pltpu.VMEM((NB,), jnp.int32),
            pltpu.VMEM((NB,), jnp.int32),
            pltpu.VMEM((NB, D), jnp.float32),
            pltpu.VMEM((NB, D), jnp.float32),
            pltpu.VMEM((NB, D), jnp.float32),
            pltpu.VMEM((2, D), jnp.float32),
            pltpu.VMEM((2 * NHOP * D,), jnp.float32),
            pltpu.VMEM((L, L), jnp.float32),
            pltpu.VMEM((NMEM + L,), jnp.float32),
            pltpu.VMEM((ROWS_PER_W,), jnp.int32),
            pltpu.VMEM((ROWS_PER_W, D), jnp.float32),
            pltpu.VMEM((L,), jnp.float32),
            pltpu.SemaphoreType.DMA,
            pltpu.SemaphoreType.DMA,
            pltpu.SemaphoreType.DMA,
            pltpu.SemaphoreType.DMA,
        ],
    )
    return f(cidx, d1idx, d2idx, rels, prot, rele, gam, bet)


def _tc_body(f1_ref, f2_ref, wagg_ref, bagg_ref, wrel_ref, brel_ref, r_ref, out_ref):
    wagg = wagg_ref[...]
    a = jnp.tanh(jnp.dot(f1_ref[...], wagg, preferred_element_type=jnp.float32)
                 + bagg_ref[...])
    b = jnp.tanh(jnp.dot(f2_ref[...], wagg, preferred_element_type=jnp.float32)
                 + bagg_ref[...])
    x = jnp.concatenate([a, b], axis=-1)
    x = jnp.tanh(jnp.dot(x, wrel_ref[...], preferred_element_type=jnp.float32)
                 + brel_ref[...])
    out_ref[...] = jnp.sum(x * r_ref[...], axis=-1)


def kernel(drug1_neighbors, drug2_neighbors, cell_neighbors, rels,
           protein_emb, rel_emb, W_agg, b_agg, W_rel, b_rel, ln_gamma, ln_beta):
    cidx = cell_neighbors.reshape(B, NB)
    d1idx = drug1_neighbors.reshape(B, NB)
    d2idx = drug2_neighbors.reshape(B, NB)
    feat1, feat2, r = _sc_call(cidx, d1idx, d2idx, rels, protein_emb, rel_emb,
                               ln_gamma, ln_beta)
    score = pl.pallas_call(
        _tc_body,
        out_shape=jax.ShapeDtypeStruct((B,), jnp.float32),
    )(feat1, feat2, W_agg, b_agg.reshape(1, D), W_rel, b_rel.reshape(1, D), r)
    return score
